# baseline (speedup 1.0000x reference)
import functools
import jax
import jax.numpy as jnp
from jax import lax
from jax.experimental import pallas as pl
from jax.experimental.pallas import tpu as pltpu
from jax.experimental.pallas import tpu_sc as plsc

B = 16384
N_ACTIONS = 1000000
DIM = 32
NC = 2
NS = 16
NW = NC * NS
BPW = B // NW

RBLK = 16384
NSTEP = (N_ACTIONS + RBLK - 1) // RBLK


def _tc_body(rows_ref, ctx_ref, qry_ref, w_ref, t_ref, d_ref):
    wa = w_ref[2, :]
    t_ref[...] = jnp.sum(rows_ref[...] * wa[None, :], axis=1)

    @pl.when(pl.program_id(0) == 0)
    def _():
        wc = w_ref[0, :]
        wq = w_ref[1, :]
        bias = w_ref[3, 0]
        d_ref[...] = (jnp.sum(ctx_ref[...] * wc[None, :], axis=1)
                      + jnp.sum(qry_ref[...] * wq[None, :], axis=1) + bias)


def _sc_body(t_hbm, act_hbm, d_hbm, out_hbm, idx_v, tv_v, dv_v, out_v, sem):
    wid = lax.axis_index("s") * NC + lax.axis_index("c")
    base = wid * BPW
    pltpu.sync_copy(act_hbm.at[pl.ds(base, BPW)], idx_v)
    gather = pltpu.async_copy(t_hbm.at[idx_v], tv_v, sem)
    pltpu.sync_copy(d_hbm.at[pl.ds(base, BPW)], dv_v)
    gather.wait()

    @plsc.parallel_loop(0, BPW // 16, step=1, unroll=8)
    def body(i):
        out_v[pl.ds(i * 16, 16)] = (tv_v[pl.ds(i * 16, 16)]
                                    + dv_v[pl.ds(i * 16, 16)])

    pltpu.sync_copy(out_v, out_hbm.at[pl.ds(base, BPW)])


@jax.jit
def _run(context, query, action, action_list, wmat):
    t, d = pl.pallas_call(
        _tc_body,
        grid=(NSTEP,),
        in_specs=[
            pl.BlockSpec((RBLK, DIM), lambda i: (i, 0)),
            pl.BlockSpec((B, DIM), lambda i: (0, 0)),
            pl.BlockSpec((B, DIM), lambda i: (0, 0)),
            pl.BlockSpec((8, DIM), lambda i: (0, 0)),
        ],
        out_specs=[
            pl.BlockSpec((RBLK,), lambda i: (i,)),
            pl.BlockSpec((B,), lambda i: (0,)),
        ],
        out_shape=[
            jax.ShapeDtypeStruct((N_ACTIONS,), jnp.float32),
            jax.ShapeDtypeStruct((B,), jnp.float32),
        ],
    )(action_list, context, query, wmat)

    mesh = plsc.VectorSubcoreMesh(core_axis_name="c", subcore_axis_name="s",
                                  num_cores=NC, num_subcores=NS)
    f = pl.kernel(
        _sc_body,
        out_type=jax.ShapeDtypeStruct((B,), jnp.float32),
        mesh=mesh,
        scratch_types=[
            pltpu.VMEM((BPW,), jnp.int32),
            pltpu.VMEM((BPW,), jnp.float32),
            pltpu.VMEM((BPW,), jnp.float32),
            pltpu.VMEM((BPW,), jnp.float32),
            pltpu.SemaphoreType.DMA,
        ],
        compiler_params=pltpu.CompilerParams(needs_layout_passes=False,
                                             use_tc_tiling_on_sc=False),
    )
    return f(t, action, d)


def kernel(context, query, action, action_list, w, b):
    wmat = jnp.zeros((8, DIM), jnp.float32)
    wmat = wmat.at[0].set(w[0:32]).at[1].set(w[32:64]).at[2].set(w[64:96])
    wmat = wmat.at[3, 0].set(b)
    return _run(context, query, action.astype(jnp.int32), action_list, wmat)


# speedup vs baseline: 7.1665x; 7.1665x over previous
import functools
import jax
import jax.numpy as jnp
from jax import lax
from jax.experimental import pallas as pl
from jax.experimental.pallas import tpu as pltpu
from jax.experimental.pallas import tpu_sc as plsc

B = 16384
N_ACTIONS = 1000000
DIM = 32
NC = 2
NS = 16
NW = NC * NS
BPW = B // NW

CBLK = 65536
NSTEP = (N_ACTIONS + CBLK - 1) // CBLK


def _tc_body(tblT_ref, ctxT_ref, qryT_ref, w_ref, t_ref, d_ref):
    wa = w_ref[2:3, :]
    t_ref[...] = jnp.dot(wa, tblT_ref[...],
                         preferred_element_type=jnp.float32)

    @pl.when(pl.program_id(0) == 0)
    def _():
        wc = w_ref[0:1, :]
        wq = w_ref[1:2, :]
        bias = w_ref[3, 0]
        d_ref[...] = (jnp.dot(wc, ctxT_ref[...],
                              preferred_element_type=jnp.float32)
                      + jnp.dot(wq, qryT_ref[...],
                                preferred_element_type=jnp.float32) + bias)


def _sc_body(t_hbm, act_hbm, d_hbm, out_hbm, idx_v, tv_v, dv_v, out_v, sem):
    wid = lax.axis_index("s") * NC + lax.axis_index("c")
    base = wid * BPW
    pltpu.sync_copy(act_hbm.at[pl.ds(base, BPW)], idx_v)
    gather = pltpu.async_copy(t_hbm.at[idx_v], tv_v, sem)
    pltpu.sync_copy(d_hbm.at[pl.ds(base, BPW)], dv_v)
    gather.wait()

    @plsc.parallel_loop(0, BPW // 16, step=1, unroll=8)
    def body(i):
        out_v[pl.ds(i * 16, 16)] = (tv_v[pl.ds(i * 16, 16)]
                                    + dv_v[pl.ds(i * 16, 16)])

    pltpu.sync_copy(out_v, out_hbm.at[pl.ds(base, BPW)])


@jax.jit
def _run(ctxT, qryT, action, tblT, wmat):
    t2, d2 = pl.pallas_call(
        _tc_body,
        grid=(NSTEP,),
        in_specs=[
            pl.BlockSpec((DIM, CBLK), lambda i: (0, i)),
            pl.BlockSpec((DIM, B), lambda i: (0, 0)),
            pl.BlockSpec((DIM, B), lambda i: (0, 0)),
            pl.BlockSpec((8, DIM), lambda i: (0, 0)),
        ],
        out_specs=[
            pl.BlockSpec((1, CBLK), lambda i: (0, i)),
            pl.BlockSpec((1, B), lambda i: (0, 0)),
        ],
        out_shape=[
            jax.ShapeDtypeStruct((1, N_ACTIONS), jnp.float32),
            jax.ShapeDtypeStruct((1, B), jnp.float32),
        ],
    )(tblT, ctxT, qryT, wmat)

    mesh = plsc.VectorSubcoreMesh(core_axis_name="c", subcore_axis_name="s",
                                  num_cores=NC, num_subcores=NS)
    f = pl.kernel(
        _sc_body,
        out_type=jax.ShapeDtypeStruct((B,), jnp.float32),
        mesh=mesh,
        scratch_types=[
            pltpu.VMEM((BPW,), jnp.int32),
            pltpu.VMEM((BPW,), jnp.float32),
            pltpu.VMEM((BPW,), jnp.float32),
            pltpu.VMEM((BPW,), jnp.float32),
            pltpu.SemaphoreType.DMA,
        ],
        compiler_params=pltpu.CompilerParams(needs_layout_passes=False,
                                             use_tc_tiling_on_sc=False),
    )
    return f(t2.reshape(N_ACTIONS), action, d2.reshape(B))


def kernel(context, query, action, action_list, w, b):
    wmat = jnp.zeros((8, DIM), jnp.float32)
    wmat = wmat.at[0].set(w[0:32]).at[1].set(w[32:64]).at[2].set(w[64:96])
    wmat = wmat.at[3, 0].set(b)
    return _run(context.T, query.T, action.astype(jnp.int32),
                action_list.T, wmat)
